# baseline (speedup 1.0000x reference)
import jax
import jax.numpy as jnp
from jax.experimental import pallas as pl
from jax.experimental.pallas import tpu as pltpu

_CHUNKS = 8


def _copy_body(pe_hbm, o_hbm, scratch, in_sems, out_sems):
    T = scratch.shape[0]
    rows = T // _CHUNKS
    for k in range(_CHUNKS):
        sl = pl.ds(k * rows, rows)
        pltpu.make_async_copy(
            pe_hbm.at[sl, :], scratch.at[sl, :], in_sems.at[k]
        ).start()
    for k in range(_CHUNKS):
        sl = pl.ds(k * rows, rows)
        pltpu.make_async_copy(
            pe_hbm.at[sl, :], scratch.at[sl, :], in_sems.at[k]
        ).wait()
        pltpu.make_async_copy(
            scratch.at[sl, :], o_hbm.at[0, sl, :], out_sems.at[k]
        ).start()
    for k in range(_CHUNKS):
        sl = pl.ds(k * rows, rows)
        pltpu.make_async_copy(
            scratch.at[sl, :], o_hbm.at[0, sl, :], out_sems.at[k]
        ).wait()


def kernel(idx, pe):
    T = idx.shape[1]
    D = pe.shape[1]
    return pl.pallas_call(
        _copy_body,
        out_shape=jax.ShapeDtypeStruct((1, T, D), pe.dtype),
        in_specs=[pl.BlockSpec(memory_space=pl.ANY)],
        out_specs=pl.BlockSpec(memory_space=pl.ANY),
        scratch_shapes=[
            pltpu.VMEM((T, D), pe.dtype),
            pltpu.SemaphoreType.DMA((_CHUNKS,)),
            pltpu.SemaphoreType.DMA((_CHUNKS,)),
        ],
    )(pe)


# speedup vs baseline: 1.0321x; 1.0321x over previous
import jax
import jax.numpy as jnp
from jax.experimental import pallas as pl
from jax.experimental.pallas import tpu as pltpu

_CHUNKS = 2


def _copy_body(pe_hbm, o_hbm, scratch, in_sems, out_sems):
    T = scratch.shape[0]
    rows = T // _CHUNKS
    for k in range(_CHUNKS):
        sl = pl.ds(k * rows, rows)
        pltpu.make_async_copy(
            pe_hbm.at[sl, :], scratch.at[sl, :], in_sems.at[k]
        ).start()
    for k in range(_CHUNKS):
        sl = pl.ds(k * rows, rows)
        pltpu.make_async_copy(
            pe_hbm.at[sl, :], scratch.at[sl, :], in_sems.at[k]
        ).wait()
        pltpu.make_async_copy(
            scratch.at[sl, :], o_hbm.at[0, sl, :], out_sems.at[k]
        ).start()
    for k in range(_CHUNKS):
        sl = pl.ds(k * rows, rows)
        pltpu.make_async_copy(
            scratch.at[sl, :], o_hbm.at[0, sl, :], out_sems.at[k]
        ).wait()


def kernel(idx, pe):
    T = idx.shape[1]
    D = pe.shape[1]
    return pl.pallas_call(
        _copy_body,
        out_shape=jax.ShapeDtypeStruct((1, T, D), pe.dtype),
        in_specs=[pl.BlockSpec(memory_space=pl.ANY)],
        out_specs=pl.BlockSpec(memory_space=pl.ANY),
        scratch_shapes=[
            pltpu.VMEM((T, D), pe.dtype),
            pltpu.SemaphoreType.DMA((_CHUNKS,)),
            pltpu.SemaphoreType.DMA((_CHUNKS,)),
        ],
    )(pe)
